# baseline (speedup 1.0000x reference)
import jax
import jax.numpy as jnp
from jax import lax
from jax.experimental import pallas as pl
from jax.experimental.pallas import tpu as pltpu
from jax.experimental.pallas import tpu_sc as plsc

N = 10000
C = 128
E_HALF = 160000
NSC = 2
NTILE = 16
CHUNK = 128
TPT = 80
EPH = NTILE * TPT * CHUNK
ROWS_H = EPH // CHUNK
DEG_PAD = 10240
DPT = DEG_PAD // NTILE
RPT = 640
G = 8
NDG = 2 * TPT // G
NSL = TPT // G


def _mm_body(x_ref, w_ref, o_ref):
    o_ref[...] = jnp.dot(x_ref[...], w_ref[...],
                         preferred_element_type=jnp.float32)


def _addrelu_body(a_ref, b_ref, o_ref):
    o_ref[...] = jnp.maximum(a_ref[...] + b_ref[...], 0.0)


def _sc_body(src2_ref, dst2_ref, w2_ref, xw_ref, out_ref,
             acc_sh, deg_sh, dis_v, dstg, wg, srcg,
             sc_idx, rows2, norm_b, tmp_v,
             gsem, ssem, psem, dsem, zsem):
    c = lax.axis_index("c")
    s = lax.axis_index("s")
    r0 = s * RPT
    t0 = s * DPT
    zbase = (NTILE - 1) * RPT

    with jax.named_scope("ph_zero"):
        def zrow(i, carry):
            for k in range(C // 16):
                rows2[0, i, pl.ds(k * 16, 16)] = jnp.zeros((16,), jnp.float32)
            return carry
        lax.fori_loop(0, CHUNK, zrow, 0)
        zsrc = rows2.at[0]

        @pl.when(s < NTILE - 1)
        def _():
            for q in range(RPT // CHUNK):
                pltpu.async_copy(zsrc, acc_sh.at[pl.ds(r0 + q * CHUNK, CHUNK)],
                                 zsem)

        @pl.when(s == NTILE - 1)
        def _():
            for q in range(3):
                pltpu.async_copy(zsrc,
                                 acc_sh.at[pl.ds(zbase + q * CHUNK, CHUNK)],
                                 zsem)
            pltpu.async_copy(zsrc.at[pl.ds(0, 16)],
                             acc_sh.at[pl.ds(zbase + 3 * CHUNK, 16)], zsem)

        def ztmp(i, carry):
            tmp_v[pl.ds(i * 16, 16)] = jnp.zeros((16,), jnp.float32)
            return carry
        lax.fori_loop(0, DPT // 16, ztmp, 0)
        pltpu.sync_copy(tmp_v, deg_sh.at[pl.ds(t0, DPT)])
        plsc.subcore_barrier()

    own_r = (c * NTILE + s) * TPT
    oth_r = ((1 - c) * NTILE + s) * TPT

    with jax.named_scope("ph_deg"):
        def drow(g):
            half = g // (NDG // 2)
            rem = g - half * (NDG // 2)
            return own_r * (1 - half) + oth_r * half + rem * G

        pltpu.sync_copy(dst2_ref.at[pl.ds(own_r, G)], dstg.at[0])
        pltpu.sync_copy(w2_ref.at[pl.ds(own_r, G)], wg.at[0])

        def dgroup(g, carry):
            b = g % 2
            nb2 = 1 - b

            @pl.when(g > 0)
            def _():
                for k in range(G):
                    pltpu.make_async_copy(
                        wg.at[nb2, k], deg_sh.at[dstg.at[nb2, k]],
                        dsem).wait()
                pltpu.make_async_copy(dst2_ref.at[pl.ds(0, G)],
                                      dstg.at[b], psem).wait()
                pltpu.make_async_copy(w2_ref.at[pl.ds(0, G)],
                                      wg.at[b], psem).wait()

            @pl.when(g < NDG - 1)
            def _():
                row = drow(g + 1)
                pltpu.async_copy(dst2_ref.at[pl.ds(row, G)], dstg.at[nb2],
                                 psem)
                pltpu.async_copy(w2_ref.at[pl.ds(row, G)], wg.at[nb2], psem)

            for k in range(G):
                pltpu.async_copy(wg.at[b, k], deg_sh.at[dstg.at[b, k]],
                                 dsem, add=True)
            return carry
        lax.fori_loop(0, NDG, dgroup, 0)
        bl = (NDG - 1) % 2
        for k in range(G):
            pltpu.make_async_copy(wg.at[bl, k], deg_sh.at[dstg.at[bl, k]],
                                  dsem).wait()
        plsc.subcore_barrier()

    with jax.named_scope("ph_newton"):
        pltpu.sync_copy(deg_sh.at[pl.ds(t0, DPT)], tmp_v)
        for k in range(DPT // 16):
            d = tmp_v[pl.ds(k * 16, 16)]
            bits = plsc.bitcast(d, jnp.int32)
            y = plsc.bitcast(jnp.int32(0x5F3759DF) - (bits >> 1), jnp.float32)
            for _ in range(3):
                y = y * (1.5 - 0.5 * d * y * y)
            tmp_v[pl.ds(k * 16, 16)] = jnp.where(d > 0.0, y, 0.0)
        pltpu.sync_copy(tmp_v, deg_sh.at[pl.ds(t0, DPT)])

        @pl.when(s < NTILE - 1)
        def _():
            for q in range(RPT // CHUNK):
                pltpu.make_async_copy(
                    zsrc, acc_sh.at[pl.ds(r0 + q * CHUNK, CHUNK)],
                    zsem).wait()

        @pl.when(s == NTILE - 1)
        def _():
            for q in range(3):
                pltpu.make_async_copy(
                    zsrc, acc_sh.at[pl.ds(zbase + q * CHUNK, CHUNK)],
                    zsem).wait()
            pltpu.make_async_copy(zsrc.at[pl.ds(0, 16)],
                                  acc_sh.at[pl.ds(zbase + 3 * CHUNK, 16)],
                                  zsem).wait()

        plsc.subcore_barrier()
        pltpu.sync_copy(deg_sh.at[pl.ds(0, N)], dis_v)

    with jax.named_scope("ph_main"):
        def stage_slab(sl, slot, copy):
            row = own_r + sl * G
            copy(src2_ref.at[pl.ds(row, G)], srcg.at[slot])
            copy(dst2_ref.at[pl.ds(row, G)], dstg.at[slot])
            copy(w2_ref.at[pl.ds(row, G)], wg.at[slot])

        stage_slab(0, 0, pltpu.sync_copy)
        pltpu.async_copy(xw_ref.at[srcg.at[0, 0]], rows2.at[0], gsem)

        def mslab(sl, carry):
            b2 = sl % 2
            nb2 = 1 - b2

            @pl.when(sl < NSL - 1)
            def _():
                stage_slab(sl + 1, nb2,
                           lambda a, d: pltpu.async_copy(a, d, psem))

            for k in range(G):
                b = k % 2
                nb_ = 1 - b
                pltpu.make_async_copy(xw_ref.at[srcg.at[b2, k]],
                                      rows2.at[b], gsem).wait()

                if k > 0:
                    pltpu.make_async_copy(rows2.at[nb_],
                                          acc_sh.at[sc_idx.at[nb_]],
                                          ssem).wait()
                else:
                    @pl.when(sl > 0)
                    def _():
                        pltpu.make_async_copy(rows2.at[nb_],
                                              acc_sh.at[sc_idx.at[nb_]],
                                              ssem).wait()

                if k < G - 1:
                    pltpu.async_copy(xw_ref.at[srcg.at[b2, k + 1]],
                                     rows2.at[nb_], gsem)
                else:
                    @pl.when(sl < NSL - 1)
                    def _():
                        for q in range(3):
                            pltpu.make_async_copy(
                                src2_ref.at[pl.ds(0, G)], srcg.at[nb2],
                                psem).wait()
                        pltpu.async_copy(xw_ref.at[srcg.at[nb2, 0]],
                                         rows2.at[nb_], gsem)

                for q in range(CHUNK // 16):
                    sc_idx[b, pl.ds(q * 16, 16)] = dstg[b2, k,
                                                        pl.ds(q * 16, 16)]
                for q in range(CHUNK // 16):
                    sv = srcg[b2, k, pl.ds(q * 16, 16)]
                    dv = dstg[b2, k, pl.ds(q * 16, 16)]
                    wv = wg[b2, k, pl.ds(q * 16, 16)]
                    nv = (plsc.load_gather(dis_v, [sv]) * wv
                          * plsc.load_gather(dis_v, [dv]))
                    norm_b[pl.ds(q * 16, 16)] = nv

                def scale(e, carry2):
                    nbv = plsc.load_gather(norm_b,
                                           [jnp.full((16,), e, jnp.int32)])
                    for q in range(C // 16):
                        rows2[b, e, pl.ds(q * 16, 16)] = (
                            rows2[b, e, pl.ds(q * 16, 16)] * nbv)
                    return carry2
                lax.fori_loop(0, CHUNK, scale, 0, unroll=8)

                pltpu.async_copy(rows2.at[b], acc_sh.at[sc_idx.at[b]],
                                 ssem, add=True)
            return carry
        lax.fori_loop(0, NSL, mslab, 0)
        pltpu.make_async_copy(rows2.at[1], acc_sh.at[sc_idx.at[1]],
                              ssem).wait()
        plsc.subcore_barrier()

    @pl.when(s < NTILE - 1)
    def _():
        pltpu.sync_copy(acc_sh.at[pl.ds(r0, RPT)],
                        out_ref.at[c, pl.ds(r0, RPT)])

    @pl.when(s == NTILE - 1)
    def _():
        pltpu.sync_copy(acc_sh.at[pl.ds(zbase, N - zbase)],
                        out_ref.at[c, pl.ds(zbase, N - zbase)])


def _prep_half(ei, w):
    src = ei[0].astype(jnp.int32)
    dst = ei[1].astype(jnp.int32)
    pad = EPH - E_HALF
    spread = (jnp.arange(pad, dtype=jnp.int32) * 61) % N
    return (jnp.concatenate([src, spread]),
            jnp.concatenate([dst, spread]),
            jnp.concatenate([w.astype(jnp.float32),
                             jnp.zeros((pad,), jnp.float32)]))


def kernel(x, u_edge_index, u_edge_weight, v_edge_index, v_edge_weight, W):
    su, du, wu = _prep_half(u_edge_index, u_edge_weight)
    sv, dv, wv = _prep_half(v_edge_index, v_edge_weight)
    src2d = jnp.concatenate([su, sv]).reshape(2 * ROWS_H, CHUNK)
    dst2d = jnp.concatenate([du, dv]).reshape(2 * ROWS_H, CHUNK)
    w2d = jnp.concatenate([wu, wv]).reshape(2 * ROWS_H, CHUNK)

    xw = pl.pallas_call(
        _mm_body, grid=(10,),
        in_specs=[pl.BlockSpec((1000, C), lambda i: (i, 0)),
                  pl.BlockSpec((C, C), lambda i: (0, 0))],
        out_specs=pl.BlockSpec((1000, C), lambda i: (i, 0)),
        out_shape=jax.ShapeDtypeStruct((N, C), jnp.float32))(x, W)

    mesh = plsc.VectorSubcoreMesh(core_axis_name="c", subcore_axis_name="s")
    partials = pl.kernel(
        _sc_body,
        out_type=jax.ShapeDtypeStruct((NSC, N, C), jnp.float32),
        mesh=mesh,
        compiler_params=pltpu.CompilerParams(needs_layout_passes=False),
        scratch_types=[
            pltpu.VMEM_SHARED((N, C), jnp.float32),
            pltpu.VMEM_SHARED((DEG_PAD,), jnp.float32),
            pltpu.VMEM((N,), jnp.float32),
            pltpu.VMEM((2, G, CHUNK), jnp.int32),
            pltpu.VMEM((2, G, CHUNK), jnp.float32),
            pltpu.VMEM((2, G, CHUNK), jnp.int32),
            pltpu.VMEM((2, CHUNK), jnp.int32),
            pltpu.VMEM((2, CHUNK, C), jnp.float32),
            pltpu.VMEM((CHUNK,), jnp.float32),
            pltpu.VMEM((DPT,), jnp.float32),
            pltpu.SemaphoreType.DMA,
            pltpu.SemaphoreType.DMA,
            pltpu.SemaphoreType.DMA,
            pltpu.SemaphoreType.DMA,
            pltpu.SemaphoreType.DMA,
        ])(src2d, dst2d, w2d, xw)

    return pl.pallas_call(
        _addrelu_body, grid=(10,),
        in_specs=[pl.BlockSpec((1000, C), lambda i: (i, 0)),
                  pl.BlockSpec((1000, C), lambda i: (i, 0))],
        out_specs=pl.BlockSpec((1000, C), lambda i: (i, 0)),
        out_shape=jax.ShapeDtypeStruct((N, C), jnp.float32))(
            partials[0], partials[1])


# speedup vs baseline: 1.0185x; 1.0185x over previous
import jax
import jax.numpy as jnp
from jax import lax
from jax.experimental import pallas as pl
from jax.experimental.pallas import tpu as pltpu
from jax.experimental.pallas import tpu_sc as plsc

N = 10000
C = 128
E_HALF = 160000
NSC = 2
NTILE = 16
CHUNK = 128
TPT = 80
EPH = NTILE * TPT * CHUNK
ROWS_H = EPH // CHUNK
DEG_PAD = 10240
DPT = DEG_PAD // NTILE
RPT = 640
G = 8
NDG = 2 * TPT // G
NSL = TPT // G


def _mm_body(x_ref, w_ref, o_ref):
    o_ref[...] = jnp.dot(x_ref[...], w_ref[...],
                         preferred_element_type=jnp.float32)


def _addrelu_body(a_ref, b_ref, o_ref):
    o_ref[...] = jnp.maximum(a_ref[...] + b_ref[...], 0.0)


def _sc_body(src2_ref, dst2_ref, w2_ref, xw_ref, out_ref,
             acc_sh, deg_sh, dis_v, dstg, wg, srcg,
             sc_idx, rows2, norm_b, tmp_v,
             gsem, ssem, psem, dsem, zsem):
    c = lax.axis_index("c")
    s = lax.axis_index("s")
    r0 = s * RPT
    t0 = s * DPT
    zbase = (NTILE - 1) * RPT

    with jax.named_scope("ph_zero"):
        def zrow(i, carry):
            for k in range(C // 16):
                rows2[0, i, pl.ds(k * 16, 16)] = jnp.zeros((16,), jnp.float32)
            return carry
        lax.fori_loop(0, CHUNK, zrow, 0)
        zsrc = rows2.at[0]

        @pl.when(s < NTILE - 1)
        def _():
            for q in range(RPT // CHUNK):
                pltpu.async_copy(zsrc, acc_sh.at[pl.ds(r0 + q * CHUNK, CHUNK)],
                                 zsem)

        @pl.when(s == NTILE - 1)
        def _():
            for q in range(3):
                pltpu.async_copy(zsrc,
                                 acc_sh.at[pl.ds(zbase + q * CHUNK, CHUNK)],
                                 zsem)
            pltpu.async_copy(zsrc.at[pl.ds(0, 16)],
                             acc_sh.at[pl.ds(zbase + 3 * CHUNK, 16)], zsem)

        def ztmp(i, carry):
            tmp_v[pl.ds(i * 16, 16)] = jnp.zeros((16,), jnp.float32)
            return carry
        lax.fori_loop(0, DPT // 16, ztmp, 0)
        pltpu.sync_copy(tmp_v, deg_sh.at[pl.ds(t0, DPT)])
        plsc.subcore_barrier()

    own_r = (c * NTILE + s) * TPT
    oth_r = ((1 - c) * NTILE + s) * TPT

    with jax.named_scope("ph_deg"):
        def drow(g):
            half = g // (NDG // 2)
            rem = g - half * (NDG // 2)
            return own_r * (1 - half) + oth_r * half + rem * G

        pltpu.sync_copy(dst2_ref.at[pl.ds(own_r, G)], dstg.at[0])
        pltpu.sync_copy(w2_ref.at[pl.ds(own_r, G)], wg.at[0])

        def dgroup(g, carry):
            b = g % 2
            nb2 = 1 - b

            @pl.when(g > 0)
            def _():
                for k in range(G):
                    pltpu.make_async_copy(
                        wg.at[nb2, k], deg_sh.at[dstg.at[nb2, k]],
                        dsem).wait()
                pltpu.make_async_copy(dst2_ref.at[pl.ds(0, G)],
                                      dstg.at[b], psem).wait()
                pltpu.make_async_copy(w2_ref.at[pl.ds(0, G)],
                                      wg.at[b], psem).wait()

            @pl.when(g < NDG - 1)
            def _():
                row = drow(g + 1)
                pltpu.async_copy(dst2_ref.at[pl.ds(row, G)], dstg.at[nb2],
                                 psem)
                pltpu.async_copy(w2_ref.at[pl.ds(row, G)], wg.at[nb2], psem)

            for k in range(G):
                pltpu.async_copy(wg.at[b, k], deg_sh.at[dstg.at[b, k]],
                                 dsem, add=True)
            return carry
        lax.fori_loop(0, NDG, dgroup, 0)
        bl = (NDG - 1) % 2
        for k in range(G):
            pltpu.make_async_copy(wg.at[bl, k], deg_sh.at[dstg.at[bl, k]],
                                  dsem).wait()
        plsc.subcore_barrier()

    with jax.named_scope("ph_newton"):
        pltpu.sync_copy(deg_sh.at[pl.ds(t0, DPT)], tmp_v)
        for k in range(DPT // 16):
            d = tmp_v[pl.ds(k * 16, 16)]
            bits = plsc.bitcast(d, jnp.int32)
            y = plsc.bitcast(jnp.int32(0x5F3759DF) - (bits >> 1), jnp.float32)
            for _ in range(3):
                y = y * (1.5 - 0.5 * d * y * y)
            tmp_v[pl.ds(k * 16, 16)] = jnp.where(d > 0.0, y, 0.0)
        pltpu.sync_copy(tmp_v, deg_sh.at[pl.ds(t0, DPT)])

        @pl.when(s < NTILE - 1)
        def _():
            for q in range(RPT // CHUNK):
                pltpu.make_async_copy(
                    zsrc, acc_sh.at[pl.ds(r0 + q * CHUNK, CHUNK)],
                    zsem).wait()

        @pl.when(s == NTILE - 1)
        def _():
            for q in range(3):
                pltpu.make_async_copy(
                    zsrc, acc_sh.at[pl.ds(zbase + q * CHUNK, CHUNK)],
                    zsem).wait()
            pltpu.make_async_copy(zsrc.at[pl.ds(0, 16)],
                                  acc_sh.at[pl.ds(zbase + 3 * CHUNK, 16)],
                                  zsem).wait()

        plsc.subcore_barrier()
        pltpu.sync_copy(deg_sh.at[pl.ds(0, N)], dis_v)

    with jax.named_scope("ph_main"):
        def stage_slab(sl, slot, copy):
            row = own_r + sl * G
            copy(src2_ref.at[pl.ds(row, G)], srcg.at[slot])
            copy(dst2_ref.at[pl.ds(row, G)], dstg.at[slot])
            copy(w2_ref.at[pl.ds(row, G)], wg.at[slot])

        stage_slab(0, 0, pltpu.sync_copy)
        pltpu.async_copy(xw_ref.at[srcg.at[0, 0]], rows2.at[0], gsem)

        def mslab(sl, carry):
            b2 = sl % 2
            nb2 = 1 - b2

            @pl.when(sl < NSL - 1)
            def _():
                stage_slab(sl + 1, nb2,
                           lambda a, d: pltpu.async_copy(a, d, psem))

            def mchunk(k, carry2):
                b = k % 2
                nb_ = 1 - b
                pltpu.make_async_copy(xw_ref.at[srcg.at[b2, k]],
                                      rows2.at[b], gsem).wait()

                @pl.when((sl > 0) | (k > 0))
                def _():
                    pltpu.make_async_copy(rows2.at[nb_],
                                          acc_sh.at[sc_idx.at[nb_]],
                                          ssem).wait()

                @pl.when(k < G - 1)
                def _():
                    pltpu.async_copy(xw_ref.at[srcg.at[b2, k + 1]],
                                     rows2.at[nb_], gsem)

                @pl.when((k == G - 1) & (sl < NSL - 1))
                def _():
                    for q in range(3):
                        pltpu.make_async_copy(
                            src2_ref.at[pl.ds(0, G)], srcg.at[nb2],
                            psem).wait()
                    pltpu.async_copy(xw_ref.at[srcg.at[nb2, 0]],
                                     rows2.at[nb_], gsem)

                for q in range(CHUNK // 16):
                    sc_idx[b, pl.ds(q * 16, 16)] = dstg[b2, k,
                                                        pl.ds(q * 16, 16)]
                for q in range(CHUNK // 16):
                    sv = srcg[b2, k, pl.ds(q * 16, 16)]
                    dv = dstg[b2, k, pl.ds(q * 16, 16)]
                    wv = wg[b2, k, pl.ds(q * 16, 16)]
                    nv = (plsc.load_gather(dis_v, [sv]) * wv
                          * plsc.load_gather(dis_v, [dv]))
                    norm_b[pl.ds(q * 16, 16)] = nv

                def scale(e, carry3):
                    nbv = plsc.load_gather(norm_b,
                                           [jnp.full((16,), e, jnp.int32)])
                    for q in range(C // 16):
                        rows2[b, e, pl.ds(q * 16, 16)] = (
                            rows2[b, e, pl.ds(q * 16, 16)] * nbv)
                    return carry3
                lax.fori_loop(0, CHUNK, scale, 0, unroll=4)

                pltpu.async_copy(rows2.at[b], acc_sh.at[sc_idx.at[b]],
                                 ssem, add=True)
                return carry2
            lax.fori_loop(0, G, mchunk, 0)
            return carry
        lax.fori_loop(0, NSL, mslab, 0)
        pltpu.make_async_copy(rows2.at[1], acc_sh.at[sc_idx.at[1]],
                              ssem).wait()
        plsc.subcore_barrier()

    @pl.when(s < NTILE - 1)
    def _():
        pltpu.sync_copy(acc_sh.at[pl.ds(r0, RPT)],
                        out_ref.at[c, pl.ds(r0, RPT)])

    @pl.when(s == NTILE - 1)
    def _():
        pltpu.sync_copy(acc_sh.at[pl.ds(zbase, N - zbase)],
                        out_ref.at[c, pl.ds(zbase, N - zbase)])


def _prep_half(ei, w):
    src = ei[0].astype(jnp.int32)
    dst = ei[1].astype(jnp.int32)
    pad = EPH - E_HALF
    spread = (jnp.arange(pad, dtype=jnp.int32) * 61) % N
    return (jnp.concatenate([src, spread]),
            jnp.concatenate([dst, spread]),
            jnp.concatenate([w.astype(jnp.float32),
                             jnp.zeros((pad,), jnp.float32)]))


def kernel(x, u_edge_index, u_edge_weight, v_edge_index, v_edge_weight, W):
    su, du, wu = _prep_half(u_edge_index, u_edge_weight)
    sv, dv, wv = _prep_half(v_edge_index, v_edge_weight)
    src2d = jnp.concatenate([su, sv]).reshape(2 * ROWS_H, CHUNK)
    dst2d = jnp.concatenate([du, dv]).reshape(2 * ROWS_H, CHUNK)
    w2d = jnp.concatenate([wu, wv]).reshape(2 * ROWS_H, CHUNK)

    xw = pl.pallas_call(
        _mm_body, grid=(10,),
        in_specs=[pl.BlockSpec((1000, C), lambda i: (i, 0)),
                  pl.BlockSpec((C, C), lambda i: (0, 0))],
        out_specs=pl.BlockSpec((1000, C), lambda i: (i, 0)),
        out_shape=jax.ShapeDtypeStruct((N, C), jnp.float32))(x, W)

    mesh = plsc.VectorSubcoreMesh(core_axis_name="c", subcore_axis_name="s")
    partials = pl.kernel(
        _sc_body,
        out_type=jax.ShapeDtypeStruct((NSC, N, C), jnp.float32),
        mesh=mesh,
        compiler_params=pltpu.CompilerParams(needs_layout_passes=False),
        scratch_types=[
            pltpu.VMEM_SHARED((N, C), jnp.float32),
            pltpu.VMEM_SHARED((DEG_PAD,), jnp.float32),
            pltpu.VMEM((N,), jnp.float32),
            pltpu.VMEM((2, G, CHUNK), jnp.int32),
            pltpu.VMEM((2, G, CHUNK), jnp.float32),
            pltpu.VMEM((2, G, CHUNK), jnp.int32),
            pltpu.VMEM((2, CHUNK), jnp.int32),
            pltpu.VMEM((2, CHUNK, C), jnp.float32),
            pltpu.VMEM((CHUNK,), jnp.float32),
            pltpu.VMEM((DPT,), jnp.float32),
            pltpu.SemaphoreType.DMA,
            pltpu.SemaphoreType.DMA,
            pltpu.SemaphoreType.DMA,
            pltpu.SemaphoreType.DMA,
            pltpu.SemaphoreType.DMA,
        ])(src2d, dst2d, w2d, xw)

    return pl.pallas_call(
        _addrelu_body, grid=(10,),
        in_specs=[pl.BlockSpec((1000, C), lambda i: (i, 0)),
                  pl.BlockSpec((1000, C), lambda i: (i, 0))],
        out_specs=pl.BlockSpec((1000, C), lambda i: (i, 0)),
        out_shape=jax.ShapeDtypeStruct((N, C), jnp.float32))(
            partials[0], partials[1])


# speedup vs baseline: 1.0258x; 1.0072x over previous
import jax
import jax.numpy as jnp
from jax import lax
from jax.experimental import pallas as pl
from jax.experimental.pallas import tpu as pltpu
from jax.experimental.pallas import tpu_sc as plsc

N = 10000
C = 128
E_HALF = 160000
NSC = 2
NTILE = 16
CHUNK = 128
TPT = 80
EPH = NTILE * TPT * CHUNK
ROWS_H = EPH // CHUNK
DEG_PAD = 10240
DPT = DEG_PAD // NTILE
RPT = 640
G = 8
NDG = 2 * TPT // G


def _mm_body(x_ref, w_ref, o_ref):
    o_ref[...] = jnp.dot(x_ref[...], w_ref[...],
                         preferred_element_type=jnp.float32)


def _addrelu_body(a_ref, b_ref, o_ref):
    o_ref[...] = jnp.maximum(a_ref[...] + b_ref[...], 0.0)


def _sc_body(src_ref, dst_ref, w_ref, dst2_ref, w2_ref, xw_ref, out_ref,
             acc_sh, deg_sh, dis_sh, dis_v, dstg, wg,
             ssrc, sdst, sw, sc_idx, rows2, norm_b, tmp_v,
             gsem, ssem, psem, dsem, zsem):
    c = lax.axis_index("c")
    s = lax.axis_index("s")
    r0 = s * RPT
    t0 = s * DPT
    zbase = (NTILE - 1) * RPT

    with jax.named_scope("ph_zero"):
        def zrow(i, carry):
            for k in range(C // 16):
                rows2[0, i, pl.ds(k * 16, 16)] = jnp.zeros((16,), jnp.float32)
            return carry
        lax.fori_loop(0, CHUNK, zrow, 0)
        zsrc = rows2.at[0]

        @pl.when(s < NTILE - 1)
        def _():
            for q in range(RPT // CHUNK):
                pltpu.async_copy(zsrc, acc_sh.at[pl.ds(r0 + q * CHUNK, CHUNK)],
                                 zsem)

        @pl.when(s == NTILE - 1)
        def _():
            for q in range(3):
                pltpu.async_copy(zsrc,
                                 acc_sh.at[pl.ds(zbase + q * CHUNK, CHUNK)],
                                 zsem)
            pltpu.async_copy(zsrc.at[pl.ds(0, 16)],
                             acc_sh.at[pl.ds(zbase + 3 * CHUNK, 16)], zsem)

        def ztmp(i, carry):
            tmp_v[pl.ds(i * 16, 16)] = jnp.zeros((16,), jnp.float32)
            return carry
        lax.fori_loop(0, DPT // 16, ztmp, 0)
        pltpu.sync_copy(tmp_v, deg_sh.at[pl.ds(t0, DPT)])
        plsc.subcore_barrier()

    own_r = (c * NTILE + s) * TPT
    oth_r = ((1 - c) * NTILE + s) * TPT
    own = own_r * CHUNK

    with jax.named_scope("ph_deg"):
        def drow(g):
            half = g // (NDG // 2)
            rem = g - half * (NDG // 2)
            return own_r * (1 - half) + oth_r * half + rem * G

        pltpu.sync_copy(dst2_ref.at[pl.ds(own_r, G)], dstg.at[0])
        pltpu.sync_copy(w2_ref.at[pl.ds(own_r, G)], wg.at[0])

        def dgroup(g, carry):
            b = g % 2
            nb2 = 1 - b

            @pl.when(g > 0)
            def _():
                for k in range(G):
                    pltpu.make_async_copy(
                        wg.at[nb2, k], deg_sh.at[dstg.at[nb2, k]],
                        dsem).wait()
                pltpu.make_async_copy(dst2_ref.at[pl.ds(0, G)],
                                      dstg.at[b], psem).wait()
                pltpu.make_async_copy(w2_ref.at[pl.ds(0, G)],
                                      wg.at[b], psem).wait()

            @pl.when(g < NDG - 1)
            def _():
                row = drow(g + 1)
                pltpu.async_copy(dst2_ref.at[pl.ds(row, G)], dstg.at[nb2],
                                 psem)
                pltpu.async_copy(w2_ref.at[pl.ds(row, G)], wg.at[nb2], psem)

            for k in range(G):
                pltpu.async_copy(wg.at[b, k], deg_sh.at[dstg.at[b, k]],
                                 dsem, add=True)
            return carry
        lax.fori_loop(0, NDG, dgroup, 0)
        bl = (NDG - 1) % 2
        for k in range(G):
            pltpu.make_async_copy(wg.at[bl, k], deg_sh.at[dstg.at[bl, k]],
                                  dsem).wait()
        plsc.subcore_barrier()

    with jax.named_scope("ph_newton"):
        pltpu.sync_copy(deg_sh.at[pl.ds(t0, DPT)], tmp_v)
        for k in range(DPT // 16):
            d = tmp_v[pl.ds(k * 16, 16)]
            bits = plsc.bitcast(d, jnp.int32)
            y = plsc.bitcast(jnp.int32(0x5F3759DF) - (bits >> 1), jnp.float32)
            for _ in range(3):
                y = y * (1.5 - 0.5 * d * y * y)
            tmp_v[pl.ds(k * 16, 16)] = jnp.where(d > 0.0, y, 0.0)
        pltpu.sync_copy(tmp_v, dis_sh.at[pl.ds(t0, DPT)])

        @pl.when(s < NTILE - 1)
        def _():
            for q in range(RPT // CHUNK):
                pltpu.make_async_copy(
                    zsrc, acc_sh.at[pl.ds(r0 + q * CHUNK, CHUNK)],
                    zsem).wait()

        @pl.when(s == NTILE - 1)
        def _():
            for q in range(3):
                pltpu.make_async_copy(
                    zsrc, acc_sh.at[pl.ds(zbase + q * CHUNK, CHUNK)],
                    zsem).wait()
            pltpu.make_async_copy(zsrc.at[pl.ds(0, 16)],
                                  acc_sh.at[pl.ds(zbase + 3 * CHUNK, 16)],
                                  zsem).wait()

        plsc.subcore_barrier()
        pltpu.sync_copy(dis_sh, dis_v)

    with jax.named_scope("ph_main"):
        def stage(j, slot, copy):
            base = own + j * CHUNK
            copy(src_ref.at[pl.ds(base, CHUNK)], ssrc.at[slot])
            copy(dst_ref.at[pl.ds(base, CHUNK)], sdst.at[slot])
            copy(w_ref.at[pl.ds(base, CHUNK)], sw.at[slot])

        stage(0, 0, pltpu.sync_copy)
        pltpu.async_copy(xw_ref.at[ssrc.at[0, pl.ds(0, 64)]],
                         rows2.at[0, pl.ds(0, 64)], gsem)
        pltpu.async_copy(xw_ref.at[ssrc.at[0, pl.ds(64, 64)]],
                         rows2.at[0, pl.ds(64, 64)], gsem)
        stage(1, 1, lambda a, b_: pltpu.async_copy(a, b_, psem))

        def mchunk(j, carry):
            b = j % 2
            nb_ = 1 - b
            pltpu.make_async_copy(xw_ref.at[ssrc.at[b, pl.ds(0, 64)]],
                                  rows2.at[b, pl.ds(0, 64)], gsem).wait()
            pltpu.make_async_copy(xw_ref.at[ssrc.at[b, pl.ds(64, 64)]],
                                  rows2.at[b, pl.ds(64, 64)], gsem).wait()

            @pl.when(j > 0)
            def _():
                pltpu.make_async_copy(rows2.at[nb_],
                                      acc_sh.at[sc_idx.at[nb_]], ssem).wait()

            @pl.when(j < TPT - 1)
            def _():
                for q in range(3):
                    pltpu.make_async_copy(src_ref.at[pl.ds(0, CHUNK)],
                                          ssrc.at[nb_], psem).wait()
                pltpu.async_copy(xw_ref.at[ssrc.at[nb_, pl.ds(0, 64)]],
                                 rows2.at[nb_, pl.ds(0, 64)], gsem)
                pltpu.async_copy(xw_ref.at[ssrc.at[nb_, pl.ds(64, 64)]],
                                 rows2.at[nb_, pl.ds(64, 64)], gsem)

            for q in range(CHUNK // 16):
                sc_idx[b, pl.ds(q * 16, 16)] = sdst[b, pl.ds(q * 16, 16)]
            for q in range(CHUNK // 16):
                sv = ssrc[b, pl.ds(q * 16, 16)]
                dv = sdst[b, pl.ds(q * 16, 16)]
                wv = sw[b, pl.ds(q * 16, 16)]
                nv = (plsc.load_gather(dis_v, [sv]) * wv
                      * plsc.load_gather(dis_v, [dv]))
                norm_b[pl.ds(q * 16, 16)] = nv

            @pl.when(j < TPT - 2)
            def _():
                stage(j + 2, b, lambda a, d: pltpu.async_copy(a, d, psem))

            def scale(e, carry2):
                nbv = plsc.load_gather(norm_b,
                                       [jnp.full((16,), e, jnp.int32)])
                for q in range(C // 16):
                    rows2[b, e, pl.ds(q * 16, 16)] = (
                        rows2[b, e, pl.ds(q * 16, 16)] * nbv)
                return carry2
            lax.fori_loop(0, CHUNK, scale, 0, unroll=8)

            pltpu.async_copy(rows2.at[b], acc_sh.at[sc_idx.at[b]], ssem,
                             add=True)
            return carry
        lax.fori_loop(0, TPT, mchunk, 0)
        pltpu.make_async_copy(rows2.at[(TPT - 1) % 2],
                              acc_sh.at[sc_idx.at[(TPT - 1) % 2]],
                              ssem).wait()
        plsc.subcore_barrier()

    @pl.when(s < NTILE - 1)
    def _():
        pltpu.sync_copy(acc_sh.at[pl.ds(r0, RPT)],
                        out_ref.at[c, pl.ds(r0, RPT)])

    @pl.when(s == NTILE - 1)
    def _():
        pltpu.sync_copy(acc_sh.at[pl.ds(zbase, N - zbase)],
                        out_ref.at[c, pl.ds(zbase, N - zbase)])


def _prep_half(ei, w):
    src = ei[0].astype(jnp.int32)
    dst = ei[1].astype(jnp.int32)
    pad = EPH - E_HALF
    spread = (jnp.arange(pad, dtype=jnp.int32) * 61) % N
    return (jnp.concatenate([src, spread]),
            jnp.concatenate([dst, spread]),
            jnp.concatenate([w.astype(jnp.float32),
                             jnp.zeros((pad,), jnp.float32)]))


def kernel(x, u_edge_index, u_edge_weight, v_edge_index, v_edge_weight, W):
    su, du, wu = _prep_half(u_edge_index, u_edge_weight)
    sv, dv, wv = _prep_half(v_edge_index, v_edge_weight)
    src1d = jnp.concatenate([su, sv])
    dst1d = jnp.concatenate([du, dv])
    w1d = jnp.concatenate([wu, wv])
    dst2d = dst1d.reshape(2 * ROWS_H, CHUNK)
    w2d = w1d.reshape(2 * ROWS_H, CHUNK)

    xw = pl.pallas_call(
        _mm_body, grid=(10,),
        in_specs=[pl.BlockSpec((1000, C), lambda i: (i, 0)),
                  pl.BlockSpec((C, C), lambda i: (0, 0))],
        out_specs=pl.BlockSpec((1000, C), lambda i: (i, 0)),
        out_shape=jax.ShapeDtypeStruct((N, C), jnp.float32))(x, W)

    mesh = plsc.VectorSubcoreMesh(core_axis_name="c", subcore_axis_name="s")
    partials = pl.kernel(
        _sc_body,
        out_type=jax.ShapeDtypeStruct((NSC, N, C), jnp.float32),
        mesh=mesh,
        compiler_params=pltpu.CompilerParams(needs_layout_passes=False),
        scratch_types=[
            pltpu.VMEM_SHARED((N, C), jnp.float32),
            pltpu.VMEM_SHARED((DEG_PAD,), jnp.float32),
            pltpu.VMEM_SHARED((DEG_PAD,), jnp.float32),
            pltpu.VMEM((DEG_PAD,), jnp.float32),
            pltpu.VMEM((2, G, CHUNK), jnp.int32),
            pltpu.VMEM((2, G, CHUNK), jnp.float32),
            pltpu.VMEM((2, CHUNK), jnp.int32),
            pltpu.VMEM((2, CHUNK), jnp.int32),
            pltpu.VMEM((2, CHUNK), jnp.float32),
            pltpu.VMEM((2, CHUNK), jnp.int32),
            pltpu.VMEM((2, CHUNK, C), jnp.float32),
            pltpu.VMEM((CHUNK,), jnp.float32),
            pltpu.VMEM((DPT,), jnp.float32),
            pltpu.SemaphoreType.DMA,
            pltpu.SemaphoreType.DMA,
            pltpu.SemaphoreType.DMA,
            pltpu.SemaphoreType.DMA,
            pltpu.SemaphoreType.DMA,
        ])(src1d, dst1d, w1d, dst2d, w2d, xw)

    return pl.pallas_call(
        _addrelu_body, grid=(10,),
        in_specs=[pl.BlockSpec((1000, C), lambda i: (i, 0)),
                  pl.BlockSpec((1000, C), lambda i: (i, 0))],
        out_specs=pl.BlockSpec((1000, C), lambda i: (i, 0)),
        out_shape=jax.ShapeDtypeStruct((N, C), jnp.float32))(
            partials[0], partials[1])
